# Initial kernel scaffold; baseline (speedup 1.0000x reference)
import jax
import jax.numpy as jnp
from jax.experimental import pallas as pl


def kernel(batch_embeds, span_starts, span_widths, attn_params, score_params):
    raise NotImplementedError("write your pallas kernel here")


# speedup vs baseline: 2.0194x; 2.0194x over previous
import functools

import jax
import jax.numpy as jnp
from jax import lax
from jax.experimental import pallas as pl
from jax.experimental.pallas import tpu as pltpu

B, T, E, S, MAX_W = 16, 2048, 512, 256, 16
HID = 150
HP = 256
OUTP = 128

_F32 = jnp.float32
_PREC = lax.Precision.HIGHEST


def _mention_kernel(emb_ref, st_ref, wd_ref,
                    aW1_ref, ab1_ref, aW2_ref, ab2_ref, aW3_ref,
                    sW1_ref, sb1_ref, sW2_ref, sb2_ref, sW3_ref,
                    se_ref, sc_ref):
    emb = emb_ref[0]

    h = jnp.maximum(
        jnp.dot(emb, aW1_ref[...], preferred_element_type=_F32,
                precision=_PREC) + ab1_ref[...], 0.0)
    h = jnp.maximum(
        jnp.dot(h, aW2_ref[...], preferred_element_type=_F32,
                precision=_PREC) + ab2_ref[...], 0.0)
    attm = jnp.dot(h, aW3_ref[...], preferred_element_type=_F32,
                   precision=_PREC)
    att = attm[:, 0:1]
    ae = emb * att

    starts = st_ref[0]
    ends = starts + wd_ref[0]
    tt = lax.broadcasted_iota(jnp.int32, (T, S), 0)
    in_span = ((tt >= starts) & (tt <= ends)).astype(_F32)
    oh_start = (tt == starts).astype(_F32)
    oh_end = (tt == ends).astype(_F32)

    dn = (((0,), (0,)), ((), ()))
    start_emb = lax.dot_general(oh_start, emb, dn,
                                preferred_element_type=_F32,
                                precision=_PREC)
    end_emb = lax.dot_general(oh_end, emb, dn,
                              preferred_element_type=_F32, precision=_PREC)
    weighted = lax.dot_general(in_span, ae, dn,
                               preferred_element_type=_F32, precision=_PREC)

    se_ref[0, :, 0:E] = start_emb
    se_ref[0, :, E:2 * E] = end_emb
    se_ref[0, :, 2 * E:3 * E] = weighted

    x = jnp.concatenate([start_emb, end_emb, weighted], axis=1)
    hs = jnp.maximum(
        jnp.dot(x, sW1_ref[...], preferred_element_type=_F32,
                precision=_PREC) + sb1_ref[...], 0.0)
    hs = jnp.maximum(
        jnp.dot(hs, sW2_ref[...], preferred_element_type=_F32,
                precision=_PREC) + sb2_ref[...], 0.0)
    sc_ref[0] = jnp.dot(hs, sW3_ref[...], preferred_element_type=_F32,
                        precision=_PREC)


def _pad_mlp(W1, b1, W2, b2, W3, b3, din):
    W1p = jnp.zeros((din, HP), _F32).at[:, :HID].set(W1)
    b1p = jnp.zeros((1, HP), _F32).at[0, :HID].set(b1)
    W2p = jnp.zeros((HP, HP), _F32).at[:HID, :HID].set(W2)
    b2p = jnp.zeros((1, HP), _F32).at[0, :HID].set(b2)
    b2p = b2p.at[0, HP - 1].set(1.0)
    W3p = jnp.zeros((HP, OUTP), _F32).at[:HID, 0].set(W3[:, 0])
    W3p = W3p.at[HP - 1, 0].set(b3[0])
    return W1p, b1p, W2p, b2p, W3p


@functools.partial(jax.jit, static_argnums=())
def kernel(batch_embeds, span_starts, span_widths, attn_params, score_params):
    aW1, ab1, aW2, ab2, aW3, ab3 = attn_params
    sW1, sb1, sW2, sb2, sW3, sb3 = score_params
    aP = _pad_mlp(aW1, ab1, aW2, ab2, aW3, ab3, E)
    sP = _pad_mlp(sW1, sb1, sW2, sb2, sW3, sb3, 3 * E)

    st3 = span_starts.reshape(B, 1, S).astype(jnp.int32)
    wd3 = span_widths.reshape(B, 1, S).astype(jnp.int32)

    def _w(shape):
        return pl.BlockSpec(shape, lambda b: (0,) * len(shape))

    grid_spec = pl.GridSpec(
        grid=(B,),
        in_specs=[
            pl.BlockSpec((1, T, E), lambda b: (b, 0, 0)),
            pl.BlockSpec((1, 1, S), lambda b: (b, 0, 0)),
            pl.BlockSpec((1, 1, S), lambda b: (b, 0, 0)),
            _w((E, HP)), _w((1, HP)), _w((HP, HP)), _w((1, HP)),
            _w((HP, OUTP)),
            _w((3 * E, HP)), _w((1, HP)), _w((HP, HP)), _w((1, HP)),
            _w((HP, OUTP)),
        ],
        out_specs=[
            pl.BlockSpec((1, S, 3 * E), lambda b: (b, 0, 0)),
            pl.BlockSpec((1, S, OUTP), lambda b: (b, 0, 0)),
        ],
    )

    span_embeds, scores = pl.pallas_call(
        _mention_kernel,
        grid_spec=grid_spec,
        out_shape=[
            jax.ShapeDtypeStruct((B, S, 3 * E), _F32),
            jax.ShapeDtypeStruct((B, S, OUTP), _F32),
        ],
        compiler_params=pltpu.CompilerParams(
            dimension_semantics=("arbitrary",),
        ),
    )(batch_embeds, st3, wd3, *aP, *sP)

    return span_embeds, scores[:, :, 0:1]


# speedup vs baseline: 8.4271x; 4.1731x over previous
import functools

import jax
import jax.numpy as jnp
from jax import lax
from jax.experimental import pallas as pl
from jax.experimental.pallas import tpu as pltpu

B, T, E, S, MAX_W = 16, 2048, 512, 256, 16
HID = 150
HP = 256
OUTP = 128

_F32 = jnp.float32
_PREC = lax.Precision.DEFAULT


def _mention_kernel(emb_ref, st_ref, wd_ref,
                    aW1_ref, ab1_ref, aW2_ref, ab2_ref, aW3_ref,
                    sW1_ref, sb1_ref, sW2_ref, sb2_ref, sW3_ref,
                    se_ref, sc_ref):
    emb = emb_ref[0]

    h = jnp.maximum(
        jnp.dot(emb, aW1_ref[...], preferred_element_type=_F32,
                precision=_PREC) + ab1_ref[...], 0.0)
    h = jnp.maximum(
        jnp.dot(h, aW2_ref[...], preferred_element_type=_F32,
                precision=_PREC) + ab2_ref[...], 0.0)
    attm = jnp.dot(h, aW3_ref[...], preferred_element_type=_F32,
                   precision=_PREC)
    att = attm[:, 0:1]

    starts = st_ref[0]
    ends = starts + wd_ref[0]
    tt = lax.broadcasted_iota(jnp.int32, (T, S), 0)
    in_span = ((tt >= starts) & (tt <= ends)).astype(_F32)
    oh_start = (tt == starts).astype(_F32)
    oh_end = (tt == ends).astype(_F32)

    big = jnp.concatenate([oh_start, oh_end, in_span * att], axis=1)
    dn = (((0,), (0,)), ((), ()))
    res = lax.dot_general(big, emb, dn,
                          preferred_element_type=_F32,
                          precision=_PREC)
    start_emb = res[0:S]
    end_emb = res[S:2 * S]
    weighted = res[2 * S:3 * S]

    se_ref[0, :, 0:E] = start_emb
    se_ref[0, :, E:2 * E] = end_emb
    se_ref[0, :, 2 * E:3 * E] = weighted

    x = jnp.concatenate([start_emb, end_emb, weighted], axis=1)
    hs = jnp.maximum(
        jnp.dot(x, sW1_ref[...], preferred_element_type=_F32,
                precision=_PREC) + sb1_ref[...], 0.0)
    hs = jnp.maximum(
        jnp.dot(hs, sW2_ref[...], preferred_element_type=_F32,
                precision=_PREC) + sb2_ref[...], 0.0)
    sc_ref[0] = jnp.dot(hs, sW3_ref[...], preferred_element_type=_F32,
                        precision=_PREC)


def _pad_mlp(W1, b1, W2, b2, W3, b3, din):
    W1p = jnp.zeros((din, HP), _F32).at[:, :HID].set(W1)
    b1p = jnp.zeros((1, HP), _F32).at[0, :HID].set(b1)
    W2p = jnp.zeros((HP, HP), _F32).at[:HID, :HID].set(W2)
    b2p = jnp.zeros((1, HP), _F32).at[0, :HID].set(b2)
    b2p = b2p.at[0, HP - 1].set(1.0)
    W3p = jnp.zeros((HP, OUTP), _F32).at[:HID, 0].set(W3[:, 0])
    W3p = W3p.at[HP - 1, 0].set(b3[0])
    return W1p, b1p, W2p, b2p, W3p


@functools.partial(jax.jit, static_argnums=())
def kernel(batch_embeds, span_starts, span_widths, attn_params, score_params):
    aW1, ab1, aW2, ab2, aW3, ab3 = attn_params
    sW1, sb1, sW2, sb2, sW3, sb3 = score_params
    aP = _pad_mlp(aW1, ab1, aW2, ab2, aW3, ab3, E)
    sP = _pad_mlp(sW1, sb1, sW2, sb2, sW3, sb3, 3 * E)

    st3 = span_starts.reshape(B, 1, S).astype(jnp.int32)
    wd3 = span_widths.reshape(B, 1, S).astype(jnp.int32)

    def _w(shape):
        return pl.BlockSpec(shape, lambda b: (0,) * len(shape))

    grid_spec = pl.GridSpec(
        grid=(B,),
        in_specs=[
            pl.BlockSpec((1, T, E), lambda b: (b, 0, 0)),
            pl.BlockSpec((1, 1, S), lambda b: (b, 0, 0)),
            pl.BlockSpec((1, 1, S), lambda b: (b, 0, 0)),
            _w((E, HP)), _w((1, HP)), _w((HP, HP)), _w((1, HP)),
            _w((HP, OUTP)),
            _w((3 * E, HP)), _w((1, HP)), _w((HP, HP)), _w((1, HP)),
            _w((HP, OUTP)),
        ],
        out_specs=[
            pl.BlockSpec((1, S, 3 * E), lambda b: (b, 0, 0)),
            pl.BlockSpec((1, S, OUTP), lambda b: (b, 0, 0)),
        ],
    )

    span_embeds, scores = pl.pallas_call(
        _mention_kernel,
        grid_spec=grid_spec,
        out_shape=[
            jax.ShapeDtypeStruct((B, S, 3 * E), _F32),
            jax.ShapeDtypeStruct((B, S, OUTP), _F32),
        ],
        compiler_params=pltpu.CompilerParams(
            dimension_semantics=("arbitrary",),
        ),
    )(batch_embeds, st3, wd3, *aP, *sP)

    return span_embeds, scores[:, :, 0:1]


# speedup vs baseline: 8.4305x; 1.0004x over previous
import jax
import jax.numpy as jnp
from jax import lax
from jax.experimental import pallas as pl
from jax.experimental.pallas import tpu as pltpu

B, T, E, S, MAX_W = 16, 2048, 512, 256, 16
HID = 150
HP = 256
OUTP = 128

_F32 = jnp.float32
_BF16 = jnp.bfloat16


def _dot(a, b):
    return jnp.dot(a, b, preferred_element_type=_F32)


def _mention_kernel(emb_ref, st_ref, wd_ref,
                    aW1_ref, ab1_ref, aW2_ref, ab2_ref, aW3_ref,
                    sW1_ref, sb1_ref, sW2_ref, sb2_ref, sW3_ref,
                    se_ref, sc_ref):
    emb = emb_ref[0]
    emb_bf = emb.astype(_BF16)

    h = jnp.maximum(_dot(emb_bf, aW1_ref[...]) + ab1_ref[...], 0.0)
    h = h.astype(_BF16)
    h = jnp.maximum(_dot(h, aW2_ref[...]) + ab2_ref[...], 0.0)
    h = h.astype(_BF16)
    attm = _dot(h, aW3_ref[...])
    att = attm[:, 0:1].astype(_BF16)

    starts = st_ref[0]
    ends = starts + wd_ref[0]
    tt = lax.broadcasted_iota(jnp.int32, (T, S), 0)
    in_span = ((tt >= starts) & (tt <= ends)).astype(_BF16)
    oh_start = (tt == starts).astype(_BF16)
    oh_end = (tt == ends).astype(_BF16)

    big = jnp.concatenate([oh_start, oh_end, in_span * att], axis=1)
    dn = (((0,), (0,)), ((), ()))
    res = lax.dot_general(big, emb_bf, dn,
                          preferred_element_type=_F32)
    start_emb = res[0:S]
    end_emb = res[S:2 * S]
    weighted = res[2 * S:3 * S]

    se_ref[0, :, 0:E] = start_emb
    se_ref[0, :, E:2 * E] = end_emb
    se_ref[0, :, 2 * E:3 * E] = weighted

    x = jnp.concatenate([start_emb, end_emb, weighted], axis=1).astype(_BF16)
    hs = jnp.maximum(_dot(x, sW1_ref[...]) + sb1_ref[...], 0.0)
    hs = hs.astype(_BF16)
    hs = jnp.maximum(_dot(hs, sW2_ref[...]) + sb2_ref[...], 0.0)
    hs = hs.astype(_BF16)
    sc_ref[0] = _dot(hs, sW3_ref[...])


def _pad_mlp(W1, b1, W2, b2, W3, b3, din):
    W1p = jnp.zeros((din, HP), _F32).at[:, :HID].set(W1).astype(_BF16)
    b1p = jnp.zeros((1, HP), _F32).at[0, :HID].set(b1)
    W2p = jnp.zeros((HP, HP), _F32).at[:HID, :HID].set(W2).astype(_BF16)
    b2p = jnp.zeros((1, HP), _F32).at[0, :HID].set(b2)
    b2p = b2p.at[0, HP - 1].set(1.0)
    W3p = jnp.zeros((HP, OUTP), _F32).at[:HID, 0].set(W3[:, 0])
    W3p = W3p.at[HP - 1, 0].set(b3[0]).astype(_BF16)
    return W1p, b1p, W2p, b2p, W3p


def kernel(batch_embeds, span_starts, span_widths, attn_params, score_params):
    aW1, ab1, aW2, ab2, aW3, ab3 = attn_params
    sW1, sb1, sW2, sb2, sW3, sb3 = score_params
    aP = _pad_mlp(aW1, ab1, aW2, ab2, aW3, ab3, E)
    sP = _pad_mlp(sW1, sb1, sW2, sb2, sW3, sb3, 3 * E)

    st3 = span_starts.reshape(B, 1, S).astype(jnp.int32)
    wd3 = span_widths.reshape(B, 1, S).astype(jnp.int32)

    def _w(shape):
        return pl.BlockSpec(shape, lambda b: (0,) * len(shape))

    grid_spec = pl.GridSpec(
        grid=(B,),
        in_specs=[
            pl.BlockSpec((1, T, E), lambda b: (b, 0, 0)),
            pl.BlockSpec((1, 1, S), lambda b: (b, 0, 0)),
            pl.BlockSpec((1, 1, S), lambda b: (b, 0, 0)),
            _w((E, HP)), _w((1, HP)), _w((HP, HP)), _w((1, HP)),
            _w((HP, OUTP)),
            _w((3 * E, HP)), _w((1, HP)), _w((HP, HP)), _w((1, HP)),
            _w((HP, OUTP)),
        ],
        out_specs=[
            pl.BlockSpec((1, S, 3 * E), lambda b: (b, 0, 0)),
            pl.BlockSpec((1, S, OUTP), lambda b: (b, 0, 0)),
        ],
    )

    span_embeds, scores = pl.pallas_call(
        _mention_kernel,
        grid_spec=grid_spec,
        out_shape=[
            jax.ShapeDtypeStruct((B, S, 3 * E), _F32),
            jax.ShapeDtypeStruct((B, S, OUTP), _F32),
        ],
        compiler_params=pltpu.CompilerParams(
            dimension_semantics=("arbitrary",),
        ),
    )(batch_embeds, st3, wd3, *aP, *sP)

    return span_embeds, scores[:, :, 0:1]


# speedup vs baseline: 10.0253x; 1.1892x over previous
import jax
import jax.numpy as jnp
from jax import lax
from jax.experimental import pallas as pl
from jax.experimental.pallas import tpu as pltpu

B, T, E, S, MAX_W = 16, 2048, 512, 256, 16
HID = 150

_F32 = jnp.float32
_BF16 = jnp.bfloat16


def _dot(a, b):
    return jnp.dot(a, b, preferred_element_type=_F32)


def _mention_kernel(emb_ref, st_ref, wd_ref,
                    aW1_ref, ab1_ref, aW2_ref, ab2_ref, aW3_ref, ab3_ref,
                    sW1_ref, sb1_ref, sW2_ref, sb2_ref, sW3_ref, sb3_ref,
                    se_ref, sc_ref):
    emb = emb_ref[0]
    emb_bf = emb.astype(_BF16)

    h = jnp.maximum(_dot(emb_bf, aW1_ref[...].astype(_BF16))
                    + ab1_ref[...], 0.0).astype(_BF16)
    h = jnp.maximum(_dot(h, aW2_ref[...].astype(_BF16))
                    + ab2_ref[...], 0.0).astype(_BF16)
    att = _dot(h, aW3_ref[...].astype(_BF16)) + ab3_ref[0]
    att = att.astype(_BF16)

    starts = st_ref[0]
    ends = starts + wd_ref[0]
    tt = lax.broadcasted_iota(jnp.int32, (T, S), 0)
    in_span = ((tt >= starts) & (tt <= ends)).astype(_BF16)
    oh_start = (tt == starts).astype(_BF16)
    oh_end = (tt == ends).astype(_BF16)

    big = jnp.concatenate([oh_start, oh_end, in_span * att], axis=1)
    dn = (((0,), (0,)), ((), ()))
    res = lax.dot_general(big, emb_bf, dn,
                          preferred_element_type=_F32)
    start_emb = res[0:S]
    end_emb = res[S:2 * S]
    weighted = res[2 * S:3 * S]

    se_ref[0, :, 0:E] = start_emb
    se_ref[0, :, E:2 * E] = end_emb
    se_ref[0, :, 2 * E:3 * E] = weighted

    x = jnp.concatenate([start_emb, end_emb, weighted], axis=1).astype(_BF16)
    hs = jnp.maximum(_dot(x, sW1_ref[...].astype(_BF16))
                     + sb1_ref[...], 0.0).astype(_BF16)
    hs = jnp.maximum(_dot(hs, sW2_ref[...].astype(_BF16))
                     + sb2_ref[...], 0.0).astype(_BF16)
    dn2 = (((0,), (1,)), ((), ()))
    sc_ref[0] = (lax.dot_general(sW3_ref[...].astype(_BF16), hs, dn2,
                                 preferred_element_type=_F32)
                 + sb3_ref[0])


def kernel(batch_embeds, span_starts, span_widths, attn_params, score_params):
    aW1, ab1, aW2, ab2, aW3, ab3 = attn_params
    sW1, sb1, sW2, sb2, sW3, sb3 = score_params

    st3 = span_starts.reshape(B, 1, S).astype(jnp.int32)
    wd3 = span_widths.reshape(B, 1, S).astype(jnp.int32)

    def _w(shape):
        return pl.BlockSpec(shape, lambda b: (0,) * len(shape))

    def _s():
        return pl.BlockSpec(memory_space=pltpu.SMEM)

    grid_spec = pl.GridSpec(
        grid=(B,),
        in_specs=[
            pl.BlockSpec((1, T, E), lambda b: (b, 0, 0)),
            pl.BlockSpec((1, 1, S), lambda b: (b, 0, 0)),
            pl.BlockSpec((1, 1, S), lambda b: (b, 0, 0)),
            _w((E, HID)), _w((1, HID)), _w((HID, HID)), _w((1, HID)),
            _w((HID, 1)), _s(),
            _w((3 * E, HID)), _w((1, HID)), _w((HID, HID)), _w((1, HID)),
            _w((HID, 1)), _s(),
        ],
        out_specs=[
            pl.BlockSpec((1, S, 3 * E), lambda b: (b, 0, 0)),
            pl.BlockSpec((1, 1, S), lambda b: (b, 0, 0)),
        ],
    )

    span_embeds, scores = pl.pallas_call(
        _mention_kernel,
        grid_spec=grid_spec,
        out_shape=[
            jax.ShapeDtypeStruct((B, S, 3 * E), _F32),
            jax.ShapeDtypeStruct((B, 1, S), _F32),
        ],
        compiler_params=pltpu.CompilerParams(
            dimension_semantics=("arbitrary",),
        ),
    )(batch_embeds, st3, wd3,
      aW1, ab1.reshape(1, HID), aW2, ab2.reshape(1, HID), aW3, ab3,
      sW1, sb1.reshape(1, HID), sW2, sb2.reshape(1, HID), sW3, sb3)

    return span_embeds, scores.reshape(B, S, 1)


# speedup vs baseline: 10.2327x; 1.0207x over previous
import jax
import jax.numpy as jnp
from jax import lax
from jax.experimental import pallas as pl
from jax.experimental.pallas import tpu as pltpu

B, T, E, S, MAX_W = 16, 2048, 512, 256, 16
HID = 150

_F32 = jnp.float32
_BF16 = jnp.bfloat16


def _dot(a, b):
    return jnp.dot(a, b, preferred_element_type=_F32)


RPB = 2


def _mention_kernel(emb_ref, st_ref, wd_ref,
                    aW1_ref, ab1_ref, aW2_ref, ab2_ref, aW3_ref, ab3_ref,
                    sW1_ref, sb1_ref, sW2_ref, sb2_ref, sW3_ref, sb3_ref,
                    se_ref, sc_ref):
    aW1 = aW1_ref[...].astype(_BF16)
    aW2 = aW2_ref[...].astype(_BF16)
    aW3 = aW3_ref[...].astype(_BF16)
    sW1 = sW1_ref[...].astype(_BF16)
    sW2 = sW2_ref[...].astype(_BF16)
    sW3 = sW3_ref[...].astype(_BF16)
    tt = lax.broadcasted_iota(jnp.int32, (T, S), 0)
    dn = (((0,), (0,)), ((), ()))
    dn2 = (((0,), (1,)), ((), ()))

    for i in range(RPB):
        emb_bf = emb_ref[i].astype(_BF16)

        h = jnp.maximum(_dot(emb_bf, aW1) + ab1_ref[...], 0.0).astype(_BF16)
        h = jnp.maximum(_dot(h, aW2) + ab2_ref[...], 0.0).astype(_BF16)
        att = (_dot(h, aW3) + ab3_ref[0]).astype(_BF16)

        starts = st_ref[i]
        ends = starts + wd_ref[i]
        in_span = ((tt >= starts) & (tt <= ends)).astype(_BF16)
        oh_start = (tt == starts).astype(_BF16)
        oh_end = (tt == ends).astype(_BF16)

        big = jnp.concatenate([oh_start, oh_end, in_span * att],
                              axis=1)
        res = lax.dot_general(big, emb_bf, dn,
                              preferred_element_type=_F32)
        start_emb = res[0:S]
        end_emb = res[S:2 * S]
        weighted = res[2 * S:3 * S]

        se_ref[i, :, 0:E] = start_emb
        se_ref[i, :, E:2 * E] = end_emb
        se_ref[i, :, 2 * E:3 * E] = weighted

        x = jnp.concatenate([start_emb, end_emb, weighted],
                            axis=1).astype(_BF16)
        hs = jnp.maximum(_dot(x, sW1) + sb1_ref[...], 0.0).astype(_BF16)
        hs = jnp.maximum(_dot(hs, sW2) + sb2_ref[...], 0.0).astype(_BF16)
        sc_ref[i] = (lax.dot_general(sW3, hs, dn2,
                                     preferred_element_type=_F32)
                     + sb3_ref[0])


def kernel(batch_embeds, span_starts, span_widths, attn_params, score_params):
    aW1, ab1, aW2, ab2, aW3, ab3 = attn_params
    sW1, sb1, sW2, sb2, sW3, sb3 = score_params

    st3 = span_starts.reshape(B, 1, S).astype(jnp.int32)
    wd3 = span_widths.reshape(B, 1, S).astype(jnp.int32)

    def _w(shape):
        return pl.BlockSpec(shape, lambda b: (0,) * len(shape))

    def _s():
        return pl.BlockSpec(memory_space=pltpu.SMEM)

    grid_spec = pl.GridSpec(
        grid=(B // RPB,),
        in_specs=[
            pl.BlockSpec((RPB, T, E), lambda b: (b, 0, 0)),
            pl.BlockSpec((RPB, 1, S), lambda b: (b, 0, 0)),
            pl.BlockSpec((RPB, 1, S), lambda b: (b, 0, 0)),
            _w((E, HID)), _w((1, HID)), _w((HID, HID)), _w((1, HID)),
            _w((HID, 1)), _s(),
            _w((3 * E, HID)), _w((1, HID)), _w((HID, HID)), _w((1, HID)),
            _w((HID, 1)), _s(),
        ],
        out_specs=[
            pl.BlockSpec((RPB, S, 3 * E), lambda b: (b, 0, 0)),
            pl.BlockSpec((RPB, 1, S), lambda b: (b, 0, 0)),
        ],
    )

    span_embeds, scores = pl.pallas_call(
        _mention_kernel,
        grid_spec=grid_spec,
        out_shape=[
            jax.ShapeDtypeStruct((B, S, 3 * E), _F32),
            jax.ShapeDtypeStruct((B, 1, S), _F32),
        ],
        compiler_params=pltpu.CompilerParams(
            dimension_semantics=("arbitrary",),
        ),
    )(batch_embeds, st3, wd3,
      aW1, ab1.reshape(1, HID), aW2, ab2.reshape(1, HID), aW3, ab3,
      sW1, sb1.reshape(1, HID), sW2, sb2.reshape(1, HID), sW3, sb3)

    return span_embeds, scores.reshape(B, S, 1)


# speedup vs baseline: 10.5090x; 1.0270x over previous
import jax
import jax.numpy as jnp
from jax import lax
from jax.experimental import pallas as pl
from jax.experimental.pallas import tpu as pltpu

B, T, E, S, MAX_W = 16, 2048, 512, 256, 16
HID = 150

_F32 = jnp.float32
_BF16 = jnp.bfloat16


def _dot(a, b):
    return jnp.dot(a, b, preferred_element_type=_F32)


RPB = 2


def _mention_kernel(emb_ref, st_ref, wd_ref,
                    aW1_ref, ab1_ref, aW2_ref, ab2_ref, aW3_ref, ab3_ref,
                    sW1_ref, sb1_ref, sW2_ref, sb2_ref, sW3_ref, sb3_ref,
                    se_ref, sc_ref):
    aW1 = aW1_ref[...].astype(_BF16)
    aW2 = aW2_ref[...].astype(_BF16)
    aW3 = aW3_ref[...].astype(_BF16)
    sW1 = sW1_ref[...].astype(_BF16)
    sW2 = sW2_ref[...].astype(_BF16)
    sW3 = sW3_ref[...].astype(_BF16)
    tt = lax.broadcasted_iota(jnp.int32, (T, S), 0)
    dn = (((0,), (0,)), ((), ()))
    dn2 = (((0,), (1,)), ((), ()))

    emb_bf = emb_ref[...].astype(_BF16).reshape(RPB * T, E)
    h = jnp.maximum(_dot(emb_bf, aW1) + ab1_ref[...], 0.0).astype(_BF16)
    h = jnp.maximum(_dot(h, aW2) + ab2_ref[...], 0.0).astype(_BF16)
    att = (_dot(h, aW3) + ab3_ref[0]).astype(_BF16)

    results = []
    for i in range(RPB):
        starts = st_ref[i]
        ends = starts + wd_ref[i]
        in_span = ((tt >= starts) & (tt <= ends)).astype(_BF16)
        oh_start = (tt == starts).astype(_BF16)
        oh_end = (tt == ends).astype(_BF16)

        att_i = att[i * T:(i + 1) * T]
        emb_i = emb_bf[i * T:(i + 1) * T]
        big = jnp.concatenate([oh_start, oh_end, in_span * att_i],
                              axis=1)
        res = lax.dot_general(big, emb_i, dn,
                              preferred_element_type=_F32)
        start_emb = res[0:S]
        end_emb = res[S:2 * S]
        weighted = res[2 * S:3 * S]

        se_ref[i, :, 0:E] = start_emb
        se_ref[i, :, E:2 * E] = end_emb
        se_ref[i, :, 2 * E:3 * E] = weighted
        results.append((start_emb, end_emb, weighted))

    x = jnp.concatenate(
        [jnp.concatenate(r, axis=1) for r in results], axis=0).astype(_BF16)
    hs = jnp.maximum(_dot(x, sW1) + sb1_ref[...], 0.0).astype(_BF16)
    hs = jnp.maximum(_dot(hs, sW2) + sb2_ref[...], 0.0).astype(_BF16)
    sc = (lax.dot_general(sW3, hs, dn2, preferred_element_type=_F32)
          + sb3_ref[0])
    for i in range(RPB):
        sc_ref[i] = sc[:, i * S:(i + 1) * S]


def kernel(batch_embeds, span_starts, span_widths, attn_params, score_params):
    aW1, ab1, aW2, ab2, aW3, ab3 = attn_params
    sW1, sb1, sW2, sb2, sW3, sb3 = score_params

    st3 = span_starts.reshape(B, 1, S).astype(jnp.int32)
    wd3 = span_widths.reshape(B, 1, S).astype(jnp.int32)

    def _w(shape):
        return pl.BlockSpec(shape, lambda b: (0,) * len(shape))

    def _s():
        return pl.BlockSpec(memory_space=pltpu.SMEM)

    grid_spec = pl.GridSpec(
        grid=(B // RPB,),
        in_specs=[
            pl.BlockSpec((RPB, T, E), lambda b: (b, 0, 0)),
            pl.BlockSpec((RPB, 1, S), lambda b: (b, 0, 0)),
            pl.BlockSpec((RPB, 1, S), lambda b: (b, 0, 0)),
            _w((E, HID)), _w((1, HID)), _w((HID, HID)), _w((1, HID)),
            _w((HID, 1)), _s(),
            _w((3 * E, HID)), _w((1, HID)), _w((HID, HID)), _w((1, HID)),
            _w((HID, 1)), _s(),
        ],
        out_specs=[
            pl.BlockSpec((RPB, S, 3 * E), lambda b: (b, 0, 0)),
            pl.BlockSpec((RPB, 1, S), lambda b: (b, 0, 0)),
        ],
    )

    span_embeds, scores = pl.pallas_call(
        _mention_kernel,
        grid_spec=grid_spec,
        out_shape=[
            jax.ShapeDtypeStruct((B, S, 3 * E), _F32),
            jax.ShapeDtypeStruct((B, 1, S), _F32),
        ],
        compiler_params=pltpu.CompilerParams(
            dimension_semantics=("arbitrary",),
        ),
    )(batch_embeds, st3, wd3,
      aW1, ab1.reshape(1, HID), aW2, ab2.reshape(1, HID), aW3, ab3,
      sW1, sb1.reshape(1, HID), sW2, sb2.reshape(1, HID), sW3, sb3)

    return span_embeds, scores.reshape(B, S, 1)
